# baseline (speedup 1.0000x reference)
import functools

import jax
import jax.numpy as jnp
from jax import lax
from jax.experimental import pallas as pl
from jax.experimental.pallas import tpu as pltpu
from jax.experimental.pallas import tpu_sc as plsc

N_EMBED = 1000000
HDIM = 64
HPAD = 128
BATCH = 16384
FIELDS = 26
FPAD = 32
N_TOTAL = BATCH * FIELDS

NW = 32
B_PER_W = N_TOTAL // NW
CHUNK = 128
N_CHUNKS = B_PER_W // CHUNK
NBUF = 4

_mesh = plsc.VectorSubcoreMesh(core_axis_name="c", subcore_axis_name="s")


@functools.partial(
    pl.kernel,
    mesh=_mesh,
    out_type=jax.ShapeDtypeStruct((BATCH * FPAD, HPAD), jnp.float32),
    scratch_types=[
        pltpu.VMEM((N_CHUNKS, CHUNK), jnp.int32),
        pltpu.VMEM((N_CHUNKS, CHUNK), jnp.int32),
        [pltpu.VMEM((CHUNK, HPAD), jnp.float32) for _ in range(NBUF)],
        [pltpu.SemaphoreType.DMA for _ in range(NBUF)],
        [pltpu.SemaphoreType.DMA for _ in range(NBUF)],
    ],
    compiler_params=pltpu.CompilerParams(use_tc_tiling_on_sc=True),
)
def _gather_kernel(
    idx_hbm, oidx_hbm, table_hbm, out_hbm, idx_v, oidx_v, bufs, sems, osems
):
    wid = lax.axis_index("s") * 2 + lax.axis_index("c")
    pltpu.sync_copy(idx_hbm.at[wid], idx_v)
    pltpu.sync_copy(oidx_hbm.at[wid], oidx_v)

    def gather_start(g, slot):
        pltpu.async_copy(table_hbm.at[idx_v.at[g]], bufs[slot], sems[slot])

    def gather_wait(slot):
        pltpu.make_async_copy(
            table_hbm.at[idx_v.at[0]], bufs[slot], sems[slot]
        ).wait()

    def scatter_start(g, slot):
        pltpu.async_copy(bufs[slot], out_hbm.at[oidx_v.at[g]], osems[slot])

    def scatter_wait(slot):
        pltpu.make_async_copy(
            bufs[slot], out_hbm.at[oidx_v.at[0]], osems[slot]
        ).wait()

    for p in range(NBUF - 1):
        gather_start(p, p)

    def step(g, q):
        gather_wait(q)
        scatter_start(g, q)
        nxt = (q + NBUF - 1) % NBUF

        @pl.when(g >= 1)
        def _():
            scatter_wait(nxt)

        @pl.when(g + NBUF - 1 < N_CHUNKS)
        def _():
            gather_start(g + NBUF - 1, nxt)

    def body(i, carry):
        for q in range(NBUF):
            step(i * NBUF + q, q)
        return carry

    lax.fori_loop(0, N_CHUNKS // NBUF, body, 0)
    scatter_wait((N_CHUNKS - 1) % NBUF)


def kernel(indices, embs):
    idx3 = indices.astype(jnp.int32).reshape(NW, N_CHUNKS, CHUNK)
    n = jnp.arange(N_TOTAL, dtype=jnp.int32)
    oidx = ((n // FIELDS) * FPAD + n % FIELDS).reshape(NW, N_CHUNKS, CHUNK)
    table = jnp.pad(embs, ((0, 0), (0, HPAD - HDIM)))
    out = _gather_kernel(idx3, oidx, table)
    return out.reshape(BATCH, FPAD, HPAD)[:, :FIELDS, :HDIM]
